# baseline (speedup 1.0000x reference)
import functools

import jax
import jax.numpy as jnp
from jax import lax
from jax.experimental import pallas as pl
from jax.experimental.pallas import tpu as pltpu
from jax.experimental.pallas import tpu_sc as plsc

LANES = 16
NUM_WORKERS = 32
I_CHUNK = 8
U_CHUNK = 32
SLAB = 8


def _item_gather_kernel(batch, embed_dim, num_rows):
  b_per_w = batch // NUM_WORKERS
  n_chunks = b_per_w // I_CHUNK
  last_tile = num_rows // 128
  mesh = plsc.VectorSubcoreMesh(core_axis_name="c", subcore_axis_name="s")

  @functools.partial(
      pl.kernel,
      out_type=jax.ShapeDtypeStruct((batch * embed_dim,), jnp.float32),
      mesh=mesh,
      compiler_params=pltpu.CompilerParams(needs_layout_passes=False),
      scratch_types=[
          pltpu.VMEM((b_per_w + LANES,), jnp.int32),
          pltpu.VMEM((I_CHUNK, embed_dim, 128), jnp.float32),
          pltpu.VMEM((2, I_CHUNK * embed_dim), jnp.float32),
          [pltpu.SemaphoreType.DMA] * I_CHUNK,
          pltpu.SemaphoreType.DMA,
      ],
  )
  def kern(iidx_hbm, itab_hbm, tail_hbm, gat_hbm,
           iidx_v, slab_v, stage_v, sems, osem):
    wid = lax.axis_index("s") * 2 + lax.axis_index("c")
    base = wid * b_per_w

    pltpu.sync_copy(iidx_hbm.at[pl.ds(base, b_per_w)],
                    iidx_v.at[pl.ds(0, b_per_w)])

    iota16 = lax.iota(jnp.int32, LANES)

    def fire(l, ri):
      t = ri >> 7

      @pl.when(t >= last_tile)
      def _():
        pltpu.async_copy(tail_hbm, slab_v.at[l], sems[l])

      @pl.when(t < last_tile)
      def _():
        pltpu.async_copy(
            itab_hbm.at[:, pl.ds(t * 128, 128)], slab_v.at[l], sems[l])

    v0 = iidx_v[pl.ds(0, LANES)]
    for l in range(I_CHUNK):
      fire(l, v0[l])

    def chunk_body(c, _):
      v = iidx_v[pl.ds(c * I_CHUNK, LANES)]
      vn = iidx_v[pl.ds((c + 1) * I_CHUNK, LANES)]
      last = c == n_chunks - 1
      p = c % 2
      @pl.when(c >= 2)
      def _():
        pltpu.make_async_copy(gat_hbm.at[pl.ds(0, I_CHUNK * embed_dim)],
                              stage_v.at[p], osem).wait()

      for l in range(I_CHUNK):
        pltpu.make_async_copy(tail_hbm, slab_v.at[l], sems[l]).wait()
        lane = jnp.full((LANES,), v[l] & 127, jnp.int32)
        slot = jnp.full((LANES,), l, jnp.int32)
        for a in range(embed_dim // LANES):
          d = a * LANES + iota16
          val = plsc.load_gather(slab_v, [slot, d, lane])
          stage_v[p, pl.ds(l * embed_dim + a * LANES, LANES)] = val

        @pl.when(jnp.logical_not(last))
        def _():
          fire(l, vn[l])

      pltpu.async_copy(
          stage_v.at[p],
          gat_hbm.at[pl.ds((base + c * I_CHUNK) * embed_dim,
                           I_CHUNK * embed_dim)], osem)
      return 0

    lax.fori_loop(0, n_chunks, chunk_body, 0)
    for _ in range(2):
      pltpu.make_async_copy(gat_hbm.at[pl.ds(0, I_CHUNK * embed_dim)],
                            stage_v.at[0], osem).wait()

  return kern


def _user_dot_kernel(batch, embed_dim):
  b_per_w = batch // NUM_WORKERS
  n_chunks = b_per_w // U_CHUNK
  mesh = plsc.VectorSubcoreMesh(core_axis_name="c", subcore_axis_name="s")

  @functools.partial(
      pl.kernel,
      out_type=jax.ShapeDtypeStruct((batch,), jnp.float32),
      mesh=mesh,
      compiler_params=pltpu.CompilerParams(needs_layout_passes=False),
      scratch_types=[
          pltpu.VMEM((b_per_w,), jnp.int32),
          pltpu.VMEM((U_CHUNK * SLAB, embed_dim), jnp.float32),
          pltpu.VMEM((U_CHUNK * embed_dim,), jnp.float32),
          pltpu.VMEM((b_per_w,), jnp.float32),
          pltpu.SemaphoreType.DMA,
      ],
  )
  def kern(uidx_hbm, utab_hbm, gat_hbm, out_hbm,
           uidx_v, uslab_v, igat_v, out_v, sem):
    wid = lax.axis_index("s") * 2 + lax.axis_index("c")
    base = wid * b_per_w

    pltpu.sync_copy(uidx_hbm.at[pl.ds(base, b_per_w)], uidx_v)

    iota16 = lax.iota(jnp.int32, LANES)

    def chunk_body(c, _):
      for g in range(U_CHUNK // LANES):
        vu = uidx_v[pl.ds(c * U_CHUNK + g * LANES, LANES)]
        for l in range(LANES):
          k = g * LANES + l
          ru = vu[l]
          pltpu.async_copy(
              utab_hbm.at[pl.ds((ru >> 3) * SLAB, SLAB), :],
              uslab_v.at[pl.ds(k * SLAB, SLAB), :], sem)
      pltpu.sync_copy(
          gat_hbm.at[pl.ds((base + c * U_CHUNK) * embed_dim,
                           U_CHUNK * embed_dim)], igat_v)
      pltpu.make_async_copy(utab_hbm.at[pl.ds(0, U_CHUNK * SLAB), :],
                            uslab_v, sem).wait()

      def group_body(g, _):
        kbase = g * LANES
        ruv = uidx_v[pl.ds(c * U_CHUNK + kbase, LANES)]
        urows = (kbase + iota16) * SLAB + (ruv & (SLAB - 1))
        irow0 = kbase * embed_dim + iota16 * embed_dim
        dcol = jnp.zeros((LANES,), jnp.int32)
        acc = (plsc.load_gather(uslab_v, [urows, dcol]) *
               plsc.load_gather(igat_v, [irow0]))
        for d in range(1, embed_dim):
          dcol = jnp.full((LANES,), d, jnp.int32)
          acc = acc + (plsc.load_gather(uslab_v, [urows, dcol]) *
                       plsc.load_gather(igat_v, [irow0 + d]))
        out_v[pl.ds(c * U_CHUNK + kbase, LANES)] = acc
        return 0

      lax.fori_loop(0, U_CHUNK // LANES, group_body, 0)
      return 0

    lax.fori_loop(0, n_chunks, chunk_body, 0)

    pltpu.sync_copy(out_v, out_hbm.at[pl.ds(base, b_per_w)])

  return kern


def kernel(user_indices, item_indices, user_weight, item_weight):
  batch = user_indices.shape[0]
  num_rows, embed_dim = user_weight.shape
  last_tile = num_rows // 128
  tail_rows = num_rows - last_tile * 128

  itab_t = item_weight.T
  tail = jnp.pad(item_weight[last_tile * 128:].T,
                 ((0, 0), (0, 128 - tail_rows)))

  k1 = _item_gather_kernel(batch, embed_dim, num_rows)
  igat = k1(item_indices.astype(jnp.int32), itab_t, tail)
  k2 = _user_dot_kernel(batch, embed_dim)
  return k2(user_indices.astype(jnp.int32), user_weight, igat)


# speedup vs baseline: 1.0010x; 1.0010x over previous
import functools

import jax
import jax.numpy as jnp
from jax import lax
from jax.experimental import pallas as pl
from jax.experimental.pallas import tpu as pltpu
from jax.experimental.pallas import tpu_sc as plsc

LANES = 16
NUM_WORKERS = 32
I_CHUNK = 8
U_CHUNK = 32
SLAB = 8


def _item_gather_kernel(batch, embed_dim, num_rows):
  b_per_w = batch // NUM_WORKERS
  n_chunks = b_per_w // I_CHUNK
  last_tile = num_rows // 128
  mesh = plsc.VectorSubcoreMesh(core_axis_name="c", subcore_axis_name="s")

  @functools.partial(
      pl.kernel,
      out_type=jax.ShapeDtypeStruct((batch * embed_dim,), jnp.float32),
      mesh=mesh,
      compiler_params=pltpu.CompilerParams(needs_layout_passes=False),
      scratch_types=[
          pltpu.VMEM((b_per_w + LANES,), jnp.int32),
          pltpu.VMEM((I_CHUNK, embed_dim, 128), jnp.float32),
          pltpu.VMEM((2, I_CHUNK * embed_dim), jnp.float32),
          [pltpu.SemaphoreType.DMA] * I_CHUNK,
          pltpu.SemaphoreType.DMA,
      ],
  )
  def kern(iidx_hbm, itab_hbm, tail_hbm, gat_hbm,
           iidx_v, slab_v, stage_v, sems, osem):
    wid = lax.axis_index("s") * 2 + lax.axis_index("c")
    base = wid * b_per_w

    pltpu.sync_copy(iidx_hbm.at[pl.ds(base, b_per_w)],
                    iidx_v.at[pl.ds(0, b_per_w)])

    iota16 = lax.iota(jnp.int32, LANES)

    def fire(l, ri):
      t = ri >> 7

      @pl.when(t >= last_tile)
      def _():
        pltpu.async_copy(tail_hbm, slab_v.at[l], sems[l])

      @pl.when(t < last_tile)
      def _():
        for dd in range(embed_dim // 8):
          pltpu.async_copy(
              itab_hbm.at[pl.ds(dd * 8, 8), pl.ds(t * 128, 128)],
              slab_v.at[l, pl.ds(dd * 8, 8), :], sems[l])

    v0 = iidx_v[pl.ds(0, LANES)]
    for l in range(I_CHUNK):
      fire(l, v0[l])

    def chunk_body(c, _):
      v = iidx_v[pl.ds(c * I_CHUNK, LANES)]
      vn = iidx_v[pl.ds((c + 1) * I_CHUNK, LANES)]
      last = c == n_chunks - 1
      p = c % 2
      @pl.when(c >= 2)
      def _():
        pltpu.make_async_copy(gat_hbm.at[pl.ds(0, I_CHUNK * embed_dim)],
                              stage_v.at[p], osem).wait()

      for l in range(I_CHUNK):
        pltpu.make_async_copy(tail_hbm, slab_v.at[l], sems[l]).wait()
        lane = jnp.full((LANES,), v[l] & 127, jnp.int32)
        slot = jnp.full((LANES,), l, jnp.int32)
        for a in range(embed_dim // LANES):
          d = a * LANES + iota16
          val = plsc.load_gather(slab_v, [slot, d, lane])
          stage_v[p, pl.ds(l * embed_dim + a * LANES, LANES)] = val

        @pl.when(jnp.logical_not(last))
        def _():
          fire(l, vn[l])

      pltpu.async_copy(
          stage_v.at[p],
          gat_hbm.at[pl.ds((base + c * I_CHUNK) * embed_dim,
                           I_CHUNK * embed_dim)], osem)
      return 0

    lax.fori_loop(0, n_chunks, chunk_body, 0)
    for _ in range(2):
      pltpu.make_async_copy(gat_hbm.at[pl.ds(0, I_CHUNK * embed_dim)],
                            stage_v.at[0], osem).wait()

  return kern


def _user_dot_kernel(batch, embed_dim):
  b_per_w = batch // NUM_WORKERS
  n_chunks = b_per_w // U_CHUNK
  mesh = plsc.VectorSubcoreMesh(core_axis_name="c", subcore_axis_name="s")

  @functools.partial(
      pl.kernel,
      out_type=jax.ShapeDtypeStruct((batch,), jnp.float32),
      mesh=mesh,
      compiler_params=pltpu.CompilerParams(needs_layout_passes=False),
      scratch_types=[
          pltpu.VMEM((b_per_w,), jnp.int32),
          pltpu.VMEM((U_CHUNK * SLAB, embed_dim), jnp.float32),
          pltpu.VMEM((U_CHUNK * embed_dim,), jnp.float32),
          pltpu.VMEM((b_per_w,), jnp.float32),
          pltpu.SemaphoreType.DMA,
      ],
  )
  def kern(uidx_hbm, utab_hbm, gat_hbm, out_hbm,
           uidx_v, uslab_v, igat_v, out_v, sem):
    wid = lax.axis_index("s") * 2 + lax.axis_index("c")
    base = wid * b_per_w

    pltpu.sync_copy(uidx_hbm.at[pl.ds(base, b_per_w)], uidx_v)

    iota16 = lax.iota(jnp.int32, LANES)

    def chunk_body(c, _):
      for g in range(U_CHUNK // LANES):
        vu = uidx_v[pl.ds(c * U_CHUNK + g * LANES, LANES)]
        for l in range(LANES):
          k = g * LANES + l
          ru = vu[l]
          pltpu.async_copy(
              utab_hbm.at[pl.ds((ru >> 3) * SLAB, SLAB), :],
              uslab_v.at[pl.ds(k * SLAB, SLAB), :], sem)
      pltpu.sync_copy(
          gat_hbm.at[pl.ds((base + c * U_CHUNK) * embed_dim,
                           U_CHUNK * embed_dim)], igat_v)
      pltpu.make_async_copy(utab_hbm.at[pl.ds(0, U_CHUNK * SLAB), :],
                            uslab_v, sem).wait()

      def group_body(g, _):
        kbase = g * LANES
        ruv = uidx_v[pl.ds(c * U_CHUNK + kbase, LANES)]
        urows = (kbase + iota16) * SLAB + (ruv & (SLAB - 1))
        irow0 = kbase * embed_dim + iota16 * embed_dim
        dcol = jnp.zeros((LANES,), jnp.int32)
        acc = (plsc.load_gather(uslab_v, [urows, dcol]) *
               plsc.load_gather(igat_v, [irow0]))
        for d in range(1, embed_dim):
          dcol = jnp.full((LANES,), d, jnp.int32)
          acc = acc + (plsc.load_gather(uslab_v, [urows, dcol]) *
                       plsc.load_gather(igat_v, [irow0 + d]))
        out_v[pl.ds(c * U_CHUNK + kbase, LANES)] = acc
        return 0

      lax.fori_loop(0, U_CHUNK // LANES, group_body, 0)
      return 0

    lax.fori_loop(0, n_chunks, chunk_body, 0)

    pltpu.sync_copy(out_v, out_hbm.at[pl.ds(base, b_per_w)])

  return kern


def kernel(user_indices, item_indices, user_weight, item_weight):
  batch = user_indices.shape[0]
  num_rows, embed_dim = user_weight.shape
  last_tile = num_rows // 128
  tail_rows = num_rows - last_tile * 128

  itab_t = item_weight.T
  tail = jnp.pad(item_weight[last_tile * 128:].T,
                 ((0, 0), (0, 128 - tail_rows)))

  k1 = _item_gather_kernel(batch, embed_dim, num_rows)
  igat = k1(item_indices.astype(jnp.int32), itab_t, tail)
  k2 = _user_dot_kernel(batch, embed_dim)
  return k2(user_indices.astype(jnp.int32), user_weight, igat)


# speedup vs baseline: 1.0021x; 1.0011x over previous
import functools

import jax
import jax.numpy as jnp
from jax import lax
from jax.experimental import pallas as pl
from jax.experimental.pallas import tpu as pltpu
from jax.experimental.pallas import tpu_sc as plsc

LANES = 16
NUM_WORKERS = 32
I_CHUNK = 8
U_CHUNK = 64
SLAB = 8


def _item_gather_kernel(batch, embed_dim, num_rows):
  b_per_w = batch // NUM_WORKERS
  n_chunks = b_per_w // I_CHUNK
  last_tile = num_rows // 128
  mesh = plsc.VectorSubcoreMesh(core_axis_name="c", subcore_axis_name="s")

  @functools.partial(
      pl.kernel,
      out_type=jax.ShapeDtypeStruct((batch * embed_dim,), jnp.float32),
      mesh=mesh,
      compiler_params=pltpu.CompilerParams(needs_layout_passes=False),
      scratch_types=[
          pltpu.VMEM((b_per_w + LANES,), jnp.int32),
          pltpu.VMEM((I_CHUNK, embed_dim, 128), jnp.float32),
          pltpu.VMEM((2, I_CHUNK * embed_dim), jnp.float32),
          [pltpu.SemaphoreType.DMA] * I_CHUNK,
          pltpu.SemaphoreType.DMA,
      ],
  )
  def kern(iidx_hbm, itab_hbm, tail_hbm, gat_hbm,
           iidx_v, slab_v, stage_v, sems, osem):
    wid = lax.axis_index("s") * 2 + lax.axis_index("c")
    base = wid * b_per_w

    pltpu.sync_copy(iidx_hbm.at[pl.ds(base, b_per_w)],
                    iidx_v.at[pl.ds(0, b_per_w)])

    iota16 = lax.iota(jnp.int32, LANES)

    def fire(l, ri):
      t = ri >> 7

      @pl.when(t >= last_tile)
      def _():
        pltpu.async_copy(tail_hbm, slab_v.at[l], sems[l])

      @pl.when(t < last_tile)
      def _():
        for dd in range(embed_dim // 8):
          pltpu.async_copy(
              itab_hbm.at[pl.ds(dd * 8, 8), pl.ds(t * 128, 128)],
              slab_v.at[l, pl.ds(dd * 8, 8), :], sems[l])

    v0 = iidx_v[pl.ds(0, LANES)]
    for l in range(I_CHUNK):
      fire(l, v0[l])

    def chunk_body(c, _):
      v = iidx_v[pl.ds(c * I_CHUNK, LANES)]
      vn = iidx_v[pl.ds((c + 1) * I_CHUNK, LANES)]
      last = c == n_chunks - 1
      p = c % 2
      @pl.when(c >= 2)
      def _():
        pltpu.make_async_copy(gat_hbm.at[pl.ds(0, I_CHUNK * embed_dim)],
                              stage_v.at[p], osem).wait()

      for l in range(I_CHUNK):
        pltpu.make_async_copy(tail_hbm, slab_v.at[l], sems[l]).wait()
        lane = jnp.full((LANES,), v[l] & 127, jnp.int32)
        slot = jnp.full((LANES,), l, jnp.int32)
        for a in range(embed_dim // LANES):
          d = a * LANES + iota16
          val = plsc.load_gather(slab_v, [slot, d, lane])
          stage_v[p, pl.ds(l * embed_dim + a * LANES, LANES)] = val

        @pl.when(jnp.logical_not(last))
        def _():
          fire(l, vn[l])

      pltpu.async_copy(
          stage_v.at[p],
          gat_hbm.at[pl.ds((base + c * I_CHUNK) * embed_dim,
                           I_CHUNK * embed_dim)], osem)
      return 0

    lax.fori_loop(0, n_chunks, chunk_body, 0)
    for _ in range(2):
      pltpu.make_async_copy(gat_hbm.at[pl.ds(0, I_CHUNK * embed_dim)],
                            stage_v.at[0], osem).wait()

  return kern


def _user_dot_kernel(batch, embed_dim):
  b_per_w = batch // NUM_WORKERS
  n_chunks = b_per_w // U_CHUNK
  mesh = plsc.VectorSubcoreMesh(core_axis_name="c", subcore_axis_name="s")

  @functools.partial(
      pl.kernel,
      out_type=jax.ShapeDtypeStruct((batch,), jnp.float32),
      mesh=mesh,
      compiler_params=pltpu.CompilerParams(needs_layout_passes=False),
      scratch_types=[
          pltpu.VMEM((b_per_w,), jnp.int32),
          pltpu.VMEM((U_CHUNK * SLAB, embed_dim), jnp.float32),
          pltpu.VMEM((U_CHUNK * embed_dim,), jnp.float32),
          pltpu.VMEM((b_per_w,), jnp.float32),
          pltpu.SemaphoreType.DMA,
      ],
  )
  def kern(uidx_hbm, utab_hbm, gat_hbm, out_hbm,
           uidx_v, uslab_v, igat_v, out_v, sem):
    wid = lax.axis_index("s") * 2 + lax.axis_index("c")
    base = wid * b_per_w

    pltpu.sync_copy(uidx_hbm.at[pl.ds(base, b_per_w)], uidx_v)

    iota16 = lax.iota(jnp.int32, LANES)

    def chunk_body(c, _):
      for g in range(U_CHUNK // LANES):
        vu = uidx_v[pl.ds(c * U_CHUNK + g * LANES, LANES)]
        for l in range(LANES):
          k = g * LANES + l
          ru = vu[l]
          pltpu.async_copy(
              utab_hbm.at[pl.ds((ru >> 3) * SLAB, SLAB), :],
              uslab_v.at[pl.ds(k * SLAB, SLAB), :], sem)
      pltpu.sync_copy(
          gat_hbm.at[pl.ds((base + c * U_CHUNK) * embed_dim,
                           U_CHUNK * embed_dim)], igat_v)
      pltpu.make_async_copy(utab_hbm.at[pl.ds(0, U_CHUNK * SLAB), :],
                            uslab_v, sem).wait()

      def group_body(g, _):
        kbase = g * LANES
        ruv = uidx_v[pl.ds(c * U_CHUNK + kbase, LANES)]
        urows = (kbase + iota16) * SLAB + (ruv & (SLAB - 1))
        irow0 = kbase * embed_dim + iota16 * embed_dim
        dcol = jnp.zeros((LANES,), jnp.int32)
        acc = (plsc.load_gather(uslab_v, [urows, dcol]) *
               plsc.load_gather(igat_v, [irow0]))
        for d in range(1, embed_dim):
          dcol = jnp.full((LANES,), d, jnp.int32)
          acc = acc + (plsc.load_gather(uslab_v, [urows, dcol]) *
                       plsc.load_gather(igat_v, [irow0 + d]))
        out_v[pl.ds(c * U_CHUNK + kbase, LANES)] = acc
        return 0

      lax.fori_loop(0, U_CHUNK // LANES, group_body, 0)
      return 0

    lax.fori_loop(0, n_chunks, chunk_body, 0)

    pltpu.sync_copy(out_v, out_hbm.at[pl.ds(base, b_per_w)])

  return kern


def kernel(user_indices, item_indices, user_weight, item_weight):
  batch = user_indices.shape[0]
  num_rows, embed_dim = user_weight.shape
  last_tile = num_rows // 128
  tail_rows = num_rows - last_tile * 128

  itab_t = item_weight.T
  tail = jnp.pad(item_weight[last_tile * 128:].T,
                 ((0, 0), (0, 128 - tail_rows)))

  k1 = _item_gather_kernel(batch, embed_dim, num_rows)
  igat = k1(item_indices.astype(jnp.int32), itab_t, tail)
  k2 = _user_dot_kernel(batch, embed_dim)
  return k2(user_indices.astype(jnp.int32), user_weight, igat)
